# Initial kernel scaffold; baseline (speedup 1.0000x reference)
import jax
import jax.numpy as jnp
from jax.experimental import pallas as pl


def kernel(input_id, input, encoder_outputs, encoder_input_ids, hidden, attention, W_ih, W_hh, b_ih, b_hh, attn_W, attn_b, comb_W, comb_b, gen_W, gen_b, copy_W, copy_b):
    raise NotImplementedError("write your pallas kernel here")


# speedup vs baseline: 5.3639x; 5.3639x over previous
import functools

import jax
import jax.numpy as jnp
from jax import lax
from jax.experimental import pallas as pl
from jax.experimental.pallas import tpu as pltpu
from jax.experimental.pallas import tpu_sc as plsc

B = 16
L = 200
EMBED = 200
HIDDEN = 256
DEC_VOCAB = 1000
VOCAB = 2000

_F32 = jnp.float32
_PREC = jax.lax.Precision.HIGHEST


def _sigmoid(x):
    return 1.0 / (1.0 + jnp.exp(-x))


def _dense_body(iid_ref, inp_ref, enc_ref, ids_ref, h0_ref, att0_ref,
                wih_i_ref, wih_a_ref, wih_s_ref, whh_ref, bih_ref, bhh_ref,
                attnw_ref, attnb_ref, combw_a_ref, combw_h_ref, combb_ref,
                genw_ref, genb_ref, copyw_ref, copyb_ref,
                gen_out_ref, copy_out_ref, hid_out_ref, att_out_ref):
    enc = enc_ref[...]
    ids = ids_ref[...]
    iid = iid_ref[...]

    mask = (iid == ids).astype(_F32)
    msum = jnp.sum(mask, axis=1, keepdims=True)
    rou = jnp.where(msum > 0, mask / jnp.where(msum > 0, msum, 1.0), 0.0)
    sel = jnp.sum(rou[:, :, None] * enc, axis=1)

    gi = (jnp.dot(inp_ref[...], wih_i_ref[...], preferred_element_type=_F32, precision=_PREC)
          + jnp.dot(att0_ref[...], wih_a_ref[...], preferred_element_type=_F32, precision=_PREC)
          + jnp.dot(sel, wih_s_ref[...], preferred_element_type=_F32, precision=_PREC)
          + bih_ref[...])
    h0 = h0_ref[...]
    gh = jnp.dot(h0, whh_ref[...], preferred_element_type=_F32, precision=_PREC) + bhh_ref[...]
    r = _sigmoid(gi[:, 0:HIDDEN] + gh[:, 0:HIDDEN])
    z = _sigmoid(gi[:, HIDDEN:2 * HIDDEN] + gh[:, HIDDEN:2 * HIDDEN])
    n = jnp.tanh(gi[:, 2 * HIDDEN:3 * HIDDEN] + r * gh[:, 2 * HIDDEN:3 * HIDDEN])
    hnew = (1.0 - z) * n + z * h0

    attn_q = jnp.dot(hnew, attnw_ref[...], preferred_element_type=_F32, precision=_PREC) + attnb_ref[...]
    logits = jnp.sum(attn_q[:, None, :] * enc, axis=2)
    lmax = jnp.max(logits, axis=1, keepdims=True)
    le = jnp.exp(logits - lmax)
    aw = le / jnp.sum(le, axis=1, keepdims=True)
    attn_applied = jnp.sum(aw[:, :, None] * enc, axis=1)
    cur_att = jnp.tanh(
        jnp.dot(attn_applied, combw_a_ref[...], preferred_element_type=_F32, precision=_PREC)
        + jnp.dot(hnew, combw_h_ref[...], preferred_element_type=_F32, precision=_PREC)
        + combb_ref[...])

    gen = jnp.dot(cur_att, genw_ref[...], preferred_element_type=_F32, precision=_PREC) + genb_ref[...]
    cw = _sigmoid(
        jnp.dot(enc.reshape(B * L, HIDDEN), copyw_ref[...],
                preferred_element_type=_F32, precision=_PREC) + copyb_ref[...])
    cs = jnp.sum(cw.reshape(B, L, HIDDEN) * cur_att[:, None, :], axis=2)

    m = jnp.maximum(jnp.max(gen, axis=1, keepdims=True),
                    jnp.max(cs, axis=1, keepdims=True))
    eg = jnp.exp(gen - m)
    ec = jnp.exp(cs - m)
    denom = jnp.sum(eg, axis=1, keepdims=True) + jnp.sum(ec, axis=1, keepdims=True)

    gen_out_ref[...] = eg / denom
    copy_out_ref[...] = ec / denom
    hid_out_ref[...] = hnew
    att_out_ref[...] = cur_att


def _log_body(x_ref, o_ref):
    x = x_ref[...]
    o_ref[...] = jnp.where(x > 0.0, jnp.log(x), -jnp.inf)


_NC = 2
_NS = 16
_LANES = 16
_LP = 208
_NCHUNKS = _LP // _LANES


@functools.cache
def _make_sc_scatter():
    @functools.partial(
        pl.kernel,
        mesh=plsc.VectorSubcoreMesh(core_axis_name="c", subcore_axis_name="s"),
        out_type=jax.ShapeDtypeStruct((B * VOCAB,), jnp.float32),
        scratch_types=[
            pltpu.VMEM((_LP,), jnp.int32),
            pltpu.VMEM((_LP,), jnp.float32),
            pltpu.VMEM((VOCAB,), jnp.float32),
        ],
        compiler_params=pltpu.CompilerParams(needs_layout_passes=False),
    )
    def _sc_scatter(ids_hbm, cs_hbm, gen_hbm, out_hbm, ids_v, cs_v, row_v):
        wid = lax.axis_index("s") * _NC + lax.axis_index("c")

        @pl.when(wid < B)
        def _():
            zi = jnp.zeros((_LANES,), jnp.int32)
            zf = jnp.zeros((_LANES,), jnp.float32)
            ids_v[pl.ds(L - 8, _LANES)] = zi
            cs_v[pl.ds(L - 8, _LANES)] = zf
            pltpu.sync_copy(ids_hbm.at[pl.ds(wid * L, L)], ids_v.at[pl.ds(0, L)])
            pltpu.sync_copy(cs_hbm.at[pl.ds(wid * L, L)], cs_v.at[pl.ds(0, L)])

            def _zero(i, carry):
                row_v[pl.ds(i * _LANES, _LANES)] = zf
                return carry

            lax.fori_loop(0, VOCAB // _LANES, _zero, 0)
            pltpu.sync_copy(gen_hbm.at[pl.ds(wid * DEC_VOCAB, DEC_VOCAB)],
                            row_v.at[pl.ds(0, DEC_VOCAB)])

            for j in range(_NCHUNKS):
                idx = ids_v[pl.ds(j * _LANES, _LANES)]
                val = cs_v[pl.ds(j * _LANES, _LANES)]
                plsc.addupdate_scatter(row_v, [idx], val)

            pltpu.sync_copy(row_v, out_hbm.at[pl.ds(wid * VOCAB, VOCAB)])

    return _sc_scatter


def kernel(input_id, input, encoder_outputs, encoder_input_ids, hidden, attention,
           W_ih, W_hh, b_ih, b_hh, attn_W, attn_b, comb_W, comb_b,
           gen_W, gen_b, copy_W, copy_b):
    inp = input[:, 0, :]
    h0 = hidden[:, 0, :]
    att0 = attention[:, 0, :]
    ids = encoder_input_ids.astype(jnp.int32)
    iid = input_id.astype(jnp.int32)

    wih_i = W_ih[:, 0:EMBED].T
    wih_a = W_ih[:, EMBED:EMBED + HIDDEN].T
    wih_s = W_ih[:, EMBED + HIDDEN:EMBED + 2 * HIDDEN].T
    whh_t = W_hh.T
    attnw_t = attn_W.T
    combw_a = comb_W[:, 0:HIDDEN].T
    combw_h = comb_W[:, HIDDEN:2 * HIDDEN].T
    genw_t = gen_W.T
    copyw_t = copy_W.T

    gen_sm, copy_sm, hnew, cur_att = pl.pallas_call(
        _dense_body,
        out_shape=[
            jax.ShapeDtypeStruct((B, DEC_VOCAB), jnp.float32),
            jax.ShapeDtypeStruct((B, L), jnp.float32),
            jax.ShapeDtypeStruct((B, HIDDEN), jnp.float32),
            jax.ShapeDtypeStruct((B, HIDDEN), jnp.float32),
        ],
    )(iid, inp, encoder_outputs, ids, h0, att0,
      wih_i, wih_a, wih_s, whh_t,
      b_ih.reshape(1, -1), b_hh.reshape(1, -1),
      attnw_t, attn_b.reshape(1, -1), combw_a, combw_h, comb_b.reshape(1, -1),
      genw_t, gen_b.reshape(1, -1), copyw_t, copy_b.reshape(1, -1))

    combined = _make_sc_scatter()(
        ids.reshape(-1), copy_sm.reshape(-1), gen_sm.reshape(-1)
    ).reshape(B, VOCAB)

    output = pl.pallas_call(
        _log_body,
        out_shape=jax.ShapeDtypeStruct((B, VOCAB), jnp.float32),
    )(combined)

    return (output, hnew[:, None, :], cur_att[:, None, :])


# speedup vs baseline: 6.5680x; 1.2245x over previous
import functools

import jax
import jax.numpy as jnp
from jax import lax
from jax.experimental import pallas as pl
from jax.experimental.pallas import tpu as pltpu
from jax.experimental.pallas import tpu_sc as plsc

B = 16
L = 200
EMBED = 200
HIDDEN = 256
DEC_VOCAB = 1000
VOCAB = 2000

_F32 = jnp.float32
_PREC = jax.lax.Precision.HIGHEST
_CONTRACT_T = (((1,), (1,)), ((), ()))


def _sigmoid(x):
    return 1.0 / (1.0 + jnp.exp(-x))


def _dotT(x, w, precision=_PREC):
    return lax.dot_general(x, w, _CONTRACT_T,
                           preferred_element_type=_F32, precision=precision)


def _dense_body(iid_ref, inp_ref, enc_ref, ids_ref, h0_ref, att0_ref,
                wih_ref, whh_ref, bih_ref, bhh_ref,
                attnw_ref, attnb_ref, combw_ref, combb_ref,
                genw_ref, genb_ref, copyw_ref, copyb_ref,
                gen_out_ref, copy_out_ref, hid_out_ref, att_out_ref):
    enc = enc_ref[...]
    ids = ids_ref[...]
    iid = iid_ref[...]

    mask = (iid == ids).astype(_F32)
    msum = jnp.sum(mask, axis=1, keepdims=True)
    rou = jnp.where(msum > 0, mask / jnp.where(msum > 0, msum, 1.0), 0.0)
    sel = jnp.sum(rou[:, :, None] * enc, axis=1)

    x = jnp.concatenate([inp_ref[:, 0, :], att0_ref[:, 0, :], sel], axis=1)
    gi = _dotT(x, wih_ref[...]) + bih_ref[...][None, :]
    h0 = h0_ref[:, 0, :]
    gh = _dotT(h0, whh_ref[...]) + bhh_ref[...][None, :]
    r = _sigmoid(gi[:, 0:HIDDEN] + gh[:, 0:HIDDEN])
    z = _sigmoid(gi[:, HIDDEN:2 * HIDDEN] + gh[:, HIDDEN:2 * HIDDEN])
    n = jnp.tanh(gi[:, 2 * HIDDEN:3 * HIDDEN] + r * gh[:, 2 * HIDDEN:3 * HIDDEN])
    hnew = (1.0 - z) * n + z * h0

    attn_q = _dotT(hnew, attnw_ref[...]) + attnb_ref[...][None, :]
    logits = jnp.sum(attn_q[:, None, :] * enc, axis=2)
    lmax = jnp.max(logits, axis=1, keepdims=True)
    le = jnp.exp(logits - lmax)
    aw = le / jnp.sum(le, axis=1, keepdims=True)
    attn_applied = jnp.sum(aw[:, :, None] * enc, axis=1)
    comb_in = jnp.concatenate([attn_applied, hnew], axis=1)
    cur_att = jnp.tanh(_dotT(comb_in, combw_ref[...]) + combb_ref[...][None, :])

    gen = _dotT(cur_att, genw_ref[...]) + genb_ref[...][None, :]
    cw = _sigmoid(_dotT(enc.reshape(B * L, HIDDEN), copyw_ref[...],
                        precision=jax.lax.Precision.DEFAULT)
                  + copyb_ref[...][None, :])
    cs = jnp.sum(cw.reshape(B, L, HIDDEN) * cur_att[:, None, :], axis=2)

    m = jnp.maximum(jnp.max(gen, axis=1, keepdims=True),
                    jnp.max(cs, axis=1, keepdims=True))
    eg = jnp.exp(gen - m)
    ec = jnp.exp(cs - m)
    denom = jnp.sum(eg, axis=1, keepdims=True) + jnp.sum(ec, axis=1, keepdims=True)

    gen_out_ref[...] = eg / denom
    copy_out_ref[...] = ec / denom
    hid_out_ref[...] = hnew[:, None, :]
    att_out_ref[...] = cur_att[:, None, :]


def _log_body(x_ref, o_ref):
    x = x_ref[...]
    o_ref[...] = jnp.where(x > 0.0, jnp.log(x), -jnp.inf)


_NC = 2
_NS = 16
_LANES = 16
_LP = 208
_NCHUNKS = _LP // _LANES


@functools.cache
def _make_sc_scatter():
    @functools.partial(
        pl.kernel,
        mesh=plsc.VectorSubcoreMesh(core_axis_name="c", subcore_axis_name="s"),
        out_type=jax.ShapeDtypeStruct((B * VOCAB,), jnp.float32),
        scratch_types=[
            pltpu.VMEM((_LP,), jnp.int32),
            pltpu.VMEM((_LP,), jnp.float32),
            pltpu.VMEM((VOCAB,), jnp.float32),
        ],
        compiler_params=pltpu.CompilerParams(needs_layout_passes=False),
    )
    def _sc_scatter(ids_hbm, cs_hbm, gen_hbm, out_hbm, ids_v, cs_v, row_v):
        wid = lax.axis_index("s") * _NC + lax.axis_index("c")

        @pl.when(wid < B)
        def _():
            zi = jnp.zeros((_LANES,), jnp.int32)
            zf = jnp.zeros((_LANES,), jnp.float32)
            ids_v[pl.ds(L - 8, _LANES)] = zi
            cs_v[pl.ds(L - 8, _LANES)] = zf
            pltpu.sync_copy(ids_hbm.at[pl.ds(wid * L, L)], ids_v.at[pl.ds(0, L)])
            pltpu.sync_copy(cs_hbm.at[pl.ds(wid * L, L)], cs_v.at[pl.ds(0, L)])

            def _zero(i, carry):
                row_v[pl.ds(i * _LANES, _LANES)] = zf
                return carry

            lax.fori_loop(0, VOCAB // _LANES, _zero, 0)
            pltpu.sync_copy(gen_hbm.at[pl.ds(wid * DEC_VOCAB, DEC_VOCAB)],
                            row_v.at[pl.ds(0, DEC_VOCAB)])

            for j in range(_NCHUNKS):
                idx = ids_v[pl.ds(j * _LANES, _LANES)]
                val = cs_v[pl.ds(j * _LANES, _LANES)]
                plsc.addupdate_scatter(row_v, [idx], val)

            pltpu.sync_copy(row_v, out_hbm.at[pl.ds(wid * VOCAB, VOCAB)])

    return _sc_scatter


def kernel(input_id, input, encoder_outputs, encoder_input_ids, hidden, attention,
           W_ih, W_hh, b_ih, b_hh, attn_W, attn_b, comb_W, comb_b,
           gen_W, gen_b, copy_W, copy_b):
    ids = encoder_input_ids.astype(jnp.int32)
    gen_sm, copy_sm, hnew, cur_att = pl.pallas_call(
        _dense_body,
        out_shape=[
            jax.ShapeDtypeStruct((B, DEC_VOCAB), jnp.float32),
            jax.ShapeDtypeStruct((B, L), jnp.float32),
            jax.ShapeDtypeStruct((B, 1, HIDDEN), jnp.float32),
            jax.ShapeDtypeStruct((B, 1, HIDDEN), jnp.float32),
        ],
    )(input_id.astype(jnp.int32), input, encoder_outputs, ids,
      hidden, attention,
      W_ih, W_hh, b_ih, b_hh, attn_W, attn_b, comb_W, comb_b,
      gen_W, gen_b, copy_W, copy_b)

    combined = _make_sc_scatter()(
        ids.reshape(-1), copy_sm.reshape(-1), gen_sm.reshape(-1))

    output = pl.pallas_call(
        _log_body,
        out_shape=jax.ShapeDtypeStruct((B * VOCAB,), jnp.float32),
    )(combined).reshape(B, VOCAB)

    return (output, hnew, cur_att)
